# Initial kernel scaffold; baseline (speedup 1.0000x reference)
import jax
import jax.numpy as jnp
from jax.experimental import pallas as pl


def kernel(lattice_coarse_values, neighbor_indices, weight):
    raise NotImplementedError("write your pallas kernel here")


# speedup vs baseline: 2.2299x; 2.2299x over previous
import functools

import jax
import jax.numpy as jnp
from jax import lax
from jax.experimental import pallas as pl
from jax.experimental.pallas import tpu as pltpu
from jax.experimental.pallas import tpu_sc as plsc

_NC = 2
_NS = 16
_NW = _NC * _NS
_LANES = 16
_C = 128


def _project_table(values, wp, m_block):
    n_coarse, d = values.shape
    n_out = wp.shape[1]

    def body(v_ref, w_ref, o_ref):
        o_ref[...] = lax.dot_general(
            v_ref[...], w_ref[...], (((1,), (0,)), ((), ())),
            preferred_element_type=jnp.float32,
            precision=lax.Precision.HIGHEST)

    return pl.pallas_call(
        body,
        grid=(n_coarse // m_block,),
        in_specs=[
            pl.BlockSpec((m_block, d), lambda i: (i, 0)),
            pl.BlockSpec((d, n_out), lambda i: (0, 0)),
        ],
        out_specs=pl.BlockSpec((m_block, n_out), lambda i: (i, 0)),
        out_shape=jax.ShapeDtypeStruct((n_coarse, n_out), jnp.float32),
    )(values, wp)


def _gather_sum(table, idx_chunks, fe, nf, n_chunks, n_pad):
    mesh = plsc.VectorSubcoreMesh(core_axis_name="c", subcore_axis_name="s")

    @functools.partial(
        pl.kernel,
        out_type=jax.ShapeDtypeStruct((n_pad, nf), jnp.float32),
        mesh=mesh,
        scratch_types=[
            pltpu.VMEM((fe, _C), jnp.int32),
            pltpu.VMEM((fe, _C, nf), jnp.float32),
            pltpu.VMEM((_C, nf), jnp.float32),
            pltpu.SemaphoreType.DMA,
        ],
        compiler_params=pltpu.CompilerParams(use_tc_tiling_on_sc=False),
    )
    def body(table_hbm, idx_hbm, out_hbm, idx_v, rows_v, acc_v, sem):
        wid = lax.axis_index("s") * _NC + lax.axis_index("c")

        @pl.loop(0, n_chunks)
        def _chunk(c):
            gchunk = wid * n_chunks + c
            pltpu.sync_copy(idx_hbm.at[gchunk], idx_v)
            copies = [
                pltpu.async_copy(table_hbm.at[idx_v.at[k]], rows_v.at[k], sem)
                for k in range(fe)
            ]
            for cp in copies:
                cp.wait()

            @pl.loop(0, _C)
            def _row(i):
                for j in range(nf // _LANES):
                    s = pl.ds(j * _LANES, _LANES)
                    v = rows_v[0, i, s]
                    for k in range(1, fe):
                        v = v + rows_v[k, i, s]
                    acc_v[i, s] = v

            pltpu.sync_copy(acc_v, out_hbm.at[pl.ds(gchunk * _C, _C)])

    return body(table, idx_chunks)


def kernel(lattice_coarse_values, neighbor_indices, weight):
    n_coarse, d = lattice_coarse_values.shape
    n_fine, fe = neighbor_indices.shape
    nf = weight.shape[1]

    wp = weight.reshape(fe, d, nf).transpose(1, 0, 2).reshape(d, fe * nf)
    p2 = _project_table(lattice_coarse_values, wp, m_block=1000)
    table = p2.reshape(n_coarse * fe, nf)

    per_round = _NW * _C
    n_chunks = -(-n_fine // per_round)
    n_pad = n_chunks * per_round
    idx32 = neighbor_indices.astype(jnp.int32)
    flat_idx = idx32 * fe + jnp.arange(fe, dtype=jnp.int32)[None, :]
    idx_t = jnp.pad(flat_idx.T, ((0, 0), (0, n_pad - n_fine)))
    idx_chunks = idx_t.reshape(fe, _NW * n_chunks, _C).transpose(1, 0, 2)

    out = _gather_sum(table, idx_chunks, fe, nf, n_chunks, n_pad)
    return out[:n_fine]
